# baseline (speedup 1.0000x reference)
import functools

import jax
import jax.numpy as jnp
from jax import lax
from jax.experimental import pallas as pl
from jax.experimental.pallas import tpu as pltpu
from jax.experimental.pallas import tpu_sc as plsc

_B = 4096
_F = 26
_ED = 8
_SPARSE = _F * _ED
_N = _B * _F
_CHUNK = 128


def _sc_gather_kernel(x_hbm, wide_hbm, deep_hbm, wide_out, deep_out,
                      idx_v, wrows, drows, sem_w, sem_d):
    info = plsc.get_sparse_core_info()
    nc = info.num_cores
    wid = lax.axis_index("s") * nc + lax.axis_index("c")
    nw = nc * info.num_subcores
    per_w = _N // nw
    n_chunks = per_w // _CHUNK

    pltpu.sync_copy(x_hbm.at[wid], idx_v)

    def body(j, carry):
        dst = pl.ds(j * _CHUNK, _CHUNK)
        cp_w = pltpu.async_copy(wide_hbm.at[idx_v.at[j]], wrows.at[dst], sem_w)
        cp_d = pltpu.async_copy(deep_hbm.at[idx_v.at[j]], drows.at[dst], sem_d)
        cp_w.wait()
        cp_d.wait()
        return carry

    lax.fori_loop(0, n_chunks, body, 0)

    base = wid * per_w
    pltpu.sync_copy(wrows, wide_out.at[pl.ds(base, per_w)])
    pltpu.sync_copy(drows, deep_out.at[pl.ds(base, per_w)])


def _sc_gather(x_flat2d, wide_table, deep_table):
    info = plsc.get_sparse_core_info()
    nw = info.num_cores * info.num_subcores
    per_w = _N // nw
    n_chunks = per_w // _CHUNK
    mesh = plsc.VectorSubcoreMesh(core_axis_name="c", subcore_axis_name="s")
    f = pl.kernel(
        _sc_gather_kernel,
        out_type=[
            jax.ShapeDtypeStruct((_N, _ED), jnp.float32),
            jax.ShapeDtypeStruct((_N, _ED), jnp.float32),
        ],
        mesh=mesh,
        scratch_types=[
            pltpu.VMEM((n_chunks, _CHUNK), jnp.int32),
            pltpu.VMEM((per_w, _ED), jnp.float32),
            pltpu.VMEM((per_w, _ED), jnp.float32),
            pltpu.SemaphoreType.DMA,
            pltpu.SemaphoreType.DMA,
        ],
        compiler_params=pltpu.CompilerParams(use_tc_tiling_on_sc=False),
    )
    return f(x_flat2d, wide_table, deep_table)


def _tc_mlp_kernel(deep_ref, wide_ref, g_ref, bta_ref,
                   w0_ref, b0_ref, w1_ref, b1_ref, w2_ref, b2_ref,
                   w3_ref, b3_ref, out_ref):
    wide_out = jnp.sum(wide_ref[...], axis=1, keepdims=True)
    h = deep_ref[...]
    mu = jnp.mean(h, axis=1, keepdims=True)
    var = jnp.mean(jnp.square(h - mu), axis=1, keepdims=True)
    h = (h - mu) * lax.rsqrt(var + 1e-5) * g_ref[...] + bta_ref[...]
    h = jnp.maximum(jnp.dot(h, w0_ref[...], preferred_element_type=jnp.float32)
                    + b0_ref[...], 0.0)
    h = jnp.maximum(jnp.dot(h, w1_ref[...], preferred_element_type=jnp.float32)
                    + b1_ref[...], 0.0)
    h = jnp.maximum(jnp.dot(h, w2_ref[...], preferred_element_type=jnp.float32)
                    + b2_ref[...], 0.0)
    dnn = jnp.sum(h * w3_ref[...], axis=1, keepdims=True) + b3_ref[...]
    out_ref[...] = jax.nn.sigmoid(wide_out + dnn)


def _tc_mlp(deep_emb, wide_emb, ln_gamma, ln_beta,
            W0, b0, W1, b1, W2, b2, W3, b3):
    bb = 1024
    grid = (_B // bb,)
    full = lambda shape: pl.BlockSpec(shape, lambda i: (0, 0))
    return pl.pallas_call(
        _tc_mlp_kernel,
        grid=grid,
        in_specs=[
            pl.BlockSpec((bb, _SPARSE), lambda i: (i, 0)),
            pl.BlockSpec((bb, _SPARSE), lambda i: (i, 0)),
            full((1, _SPARSE)),
            full((1, _SPARSE)),
            full((_SPARSE, 512)),
            full((1, 512)),
            full((512, 256)),
            full((1, 256)),
            full((256, 128)),
            full((1, 128)),
            full((1, 128)),
            full((1, 1)),
        ],
        out_specs=pl.BlockSpec((bb, 1), lambda i: (i, 0)),
        out_shape=jax.ShapeDtypeStruct((_B, 1), jnp.float32),
    )(deep_emb, wide_emb, ln_gamma.reshape(1, -1), ln_beta.reshape(1, -1),
      W0, b0.reshape(1, -1), W1, b1.reshape(1, -1), W2, b2.reshape(1, -1),
      W3.reshape(1, -1), b3.reshape(1, 1))


@jax.jit
def kernel(x, wide_table, deep_table, ln_gamma, ln_beta,
           W0, b0, W1, b1, W2, b2, W3, b3):
    x_flat2d = x.reshape(32, _N // (32 * _CHUNK), _CHUNK)
    wide_rows, deep_rows = _sc_gather(x_flat2d, wide_table, deep_table)
    wide_emb = wide_rows.reshape(_B, _SPARSE)
    deep_emb = deep_rows.reshape(_B, _SPARSE)
    return _tc_mlp(deep_emb, wide_emb, ln_gamma, ln_beta,
                   W0, b0, W1, b1, W2, b2, W3, b3)


# speedup vs baseline: 1.0150x; 1.0150x over previous
import functools

import jax
import jax.numpy as jnp
from jax import lax
from jax.experimental import pallas as pl
from jax.experimental.pallas import tpu as pltpu
from jax.experimental.pallas import tpu_sc as plsc

_B = 4096
_F = 26
_ED = 8
_SPARSE = _F * _ED
_N = _B * _F
_CHUNK = 128


def _sc_gather_kernel(x_hbm, wide_hbm, deep_hbm, wide_out, deep_out,
                      idx_v, wrows, drows, sem_w, sem_d):
    info = plsc.get_sparse_core_info()
    nc = info.num_cores
    wid = lax.axis_index("s") * nc + lax.axis_index("c")
    nw = nc * info.num_subcores
    per_w = _N // nw

    pltpu.sync_copy(x_hbm.at[wid], idx_v)
    cp_w = pltpu.async_copy(wide_hbm.at[idx_v], wrows, sem_w)
    cp_d = pltpu.async_copy(deep_hbm.at[idx_v], drows, sem_d)
    base = wid * per_w
    cp_w.wait()
    pltpu.sync_copy(wrows, wide_out.at[pl.ds(base, per_w)])
    cp_d.wait()
    pltpu.sync_copy(drows, deep_out.at[pl.ds(base, per_w)])


def _sc_gather(x_flat2d, wide_table, deep_table):
    info = plsc.get_sparse_core_info()
    nw = info.num_cores * info.num_subcores
    per_w = _N // nw
    n_chunks = per_w // _CHUNK
    mesh = plsc.VectorSubcoreMesh(core_axis_name="c", subcore_axis_name="s")
    f = pl.kernel(
        _sc_gather_kernel,
        out_type=[
            jax.ShapeDtypeStruct((_N, _ED), jnp.float32),
            jax.ShapeDtypeStruct((_N, _ED), jnp.float32),
        ],
        mesh=mesh,
        scratch_types=[
            pltpu.VMEM((per_w,), jnp.int32),
            pltpu.VMEM((per_w, _ED), jnp.float32),
            pltpu.VMEM((per_w, _ED), jnp.float32),
            pltpu.SemaphoreType.DMA,
            pltpu.SemaphoreType.DMA,
        ],
        compiler_params=pltpu.CompilerParams(use_tc_tiling_on_sc=False),
    )
    return f(x_flat2d, wide_table, deep_table)


def _tc_mlp_kernel(deep_ref, wide_ref, g_ref, bta_ref,
                   w0_ref, b0_ref, w1_ref, b1_ref, w2_ref, b2_ref,
                   w3_ref, b3_ref, out_ref):
    wide_out = jnp.sum(wide_ref[...], axis=1, keepdims=True)
    h = deep_ref[...]
    mu = jnp.mean(h, axis=1, keepdims=True)
    var = jnp.mean(jnp.square(h - mu), axis=1, keepdims=True)
    h = (h - mu) * lax.rsqrt(var + 1e-5) * g_ref[...] + bta_ref[...]
    h = jnp.maximum(jnp.dot(h, w0_ref[...], preferred_element_type=jnp.float32)
                    + b0_ref[...], 0.0)
    h = jnp.maximum(jnp.dot(h, w1_ref[...], preferred_element_type=jnp.float32)
                    + b1_ref[...], 0.0)
    h = jnp.maximum(jnp.dot(h, w2_ref[...], preferred_element_type=jnp.float32)
                    + b2_ref[...], 0.0)
    dnn = jnp.sum(h * w3_ref[...], axis=1, keepdims=True) + b3_ref[...]
    out_ref[...] = jax.nn.sigmoid(wide_out + dnn)


def _tc_mlp(deep_emb, wide_emb, ln_gamma, ln_beta,
            W0, b0, W1, b1, W2, b2, W3, b3):
    bb = 1024
    grid = (_B // bb,)
    full = lambda shape: pl.BlockSpec(shape, lambda i: (0, 0))
    return pl.pallas_call(
        _tc_mlp_kernel,
        grid=grid,
        in_specs=[
            pl.BlockSpec((bb, _SPARSE), lambda i: (i, 0)),
            pl.BlockSpec((bb, _SPARSE), lambda i: (i, 0)),
            full((1, _SPARSE)),
            full((1, _SPARSE)),
            full((_SPARSE, 512)),
            full((1, 512)),
            full((512, 256)),
            full((1, 256)),
            full((256, 128)),
            full((1, 128)),
            full((1, 128)),
            full((1, 1)),
        ],
        out_specs=pl.BlockSpec((bb, 1), lambda i: (i, 0)),
        out_shape=jax.ShapeDtypeStruct((_B, 1), jnp.float32),
    )(deep_emb, wide_emb, ln_gamma.reshape(1, -1), ln_beta.reshape(1, -1),
      W0, b0.reshape(1, -1), W1, b1.reshape(1, -1), W2, b2.reshape(1, -1),
      W3.reshape(1, -1), b3.reshape(1, 1))


@jax.jit
def kernel(x, wide_table, deep_table, ln_gamma, ln_beta,
           W0, b0, W1, b1, W2, b2, W3, b3):
    x_flat2d = x.reshape(32, _N // 32)
    wide_rows, deep_rows = _sc_gather(x_flat2d, wide_table, deep_table)
    wide_emb = wide_rows.reshape(_B, _SPARSE)
    deep_emb = deep_rows.reshape(_B, _SPARSE)
    return _tc_mlp(deep_emb, wide_emb, ln_gamma, ln_beta,
                   W0, b0, W1, b1, W2, b2, W3, b3)
